# baseline (speedup 1.0000x reference)
import jax
import jax.numpy as jnp
from jax.experimental import pallas as pl


def _copy_kernel(x_ref, o_ref):
    o_ref[...] = x_ref[...]


@jax.jit
def kernel(x_num, x_cat, num_weight, num_bias, cat_tables, cls_token):
    B = x_num.shape[0]
    num_tokens = x_num[:, :, None] * num_weight[None, :, :] + num_bias[None, :, :]
    cat_tokens = jax.vmap(lambda table, idx: jnp.take(table, idx, axis=0),
                          in_axes=(0, 1), out_axes=1)(cat_tables, x_cat)
    cls = jnp.broadcast_to(cls_token, (B, 1, cls_token.shape[-1]))
    tokens = jnp.concatenate([cls, num_tokens, cat_tokens], axis=1)
    touched = pl.pallas_call(
        _copy_kernel,
        out_shape=jax.ShapeDtypeStruct((8, 128), jnp.float32),
    )(tokens[:8, 0, :64].reshape(8, 64).repeat(2, axis=1))
    return tokens + 0.0 * touched[0, 0]


# speedup vs baseline: 1.2150x; 1.2150x over previous
import jax
import jax.numpy as jnp
from jax import lax
from jax.experimental import pallas as pl
from jax.experimental.pallas import tpu as pltpu
from jax.experimental.pallas import tpu_sc as plsc

N_NUM = 13
N_CAT = 26
VOCAB = 100000
D = 64
B = 4096
N_TOK = 1 + N_NUM + N_CAT

NUM_CORES = 2
NUM_SUBCORES = 16
NW = NUM_CORES * NUM_SUBCORES

W_IDS = 4096
NWIN = (VOCAB + W_IDS - 1) // W_IDS
TAIL0 = (NWIN - 1) * W_IDS
TAILW = 1792
NBND = 32

N_CGRP = N_CAT * (D // 8)
N_DGRP = (1 + N_NUM) * (D // 8)


def _tokenizer_kernel(xn1d_hbm, skeys1d_hbm, bnds1d_hbm, w1d_hbm, b1d_hbm,
                      cls_hbm, tt_hbm, aux_hbm, out_hbm,
                      wbuf, obuf, skeys_v, xn_v, bnd_v, prm_v,
                      ssem, osem, psem, dsem):
    core = lax.axis_index("c")
    sub = lax.axis_index("s")
    wid = sub * NUM_CORES + core

    lanes = lax.iota(jnp.int32, 16)

    ndense = jnp.where(wid < 16, 3, 4)

    def dense_body(j, _):
        h = (31 - wid) + 32 * j
        t = h // 8
        dg = h % 8
        tm1 = jnp.maximum(t - 1, 0)
        xoff = pl.multiple_of(tm1 * B, 8)
        pltpu.async_copy(xn1d_hbm.at[pl.ds(xoff, B)], xn_v, dsem)
        poff = pl.multiple_of(tm1 * D + dg * 8, 8)
        coff = pl.multiple_of(dg * 8, 8)
        pltpu.async_copy(w1d_hbm.at[pl.ds(poff, 8)], prm_v.at[pl.ds(0, 8)],
                         psem)
        pltpu.async_copy(b1d_hbm.at[pl.ds(poff, 8)], prm_v.at[pl.ds(16, 8)],
                         psem)
        pltpu.async_copy(cls_hbm.at[pl.ds(coff, 8)], prm_v.at[pl.ds(32, 8)],
                         psem)
        pltpu.make_async_copy(xn1d_hbm.at[pl.ds(xoff, B)], xn_v, dsem).wait()
        pltpu.make_async_copy(w1d_hbm.at[pl.ds(poff, 8)],
                              prm_v.at[pl.ds(0, 8)], psem).wait()
        pltpu.make_async_copy(b1d_hbm.at[pl.ds(poff, 8)],
                              prm_v.at[pl.ds(16, 8)], psem).wait()
        pltpu.make_async_copy(cls_hbm.at[pl.ds(coff, 8)],
                              prm_v.at[pl.ds(32, 8)], psem).wait()
        wv8 = prm_v[pl.ds(0, 16)]
        bv8 = prm_v[pl.ds(16, 16)]
        cv8 = prm_v[pl.ds(32, 16)]

        for dd in range(8):
            is_cls = t == 0
            s_mul = jnp.where(is_cls, 0.0, wv8[dd])
            s_add = jnp.where(is_cls, cv8[dd], bv8[dd])

            def row_body(v, _):
                sl = pl.ds(v * 16, 16)
                obuf[dd, sl] = xn_v[sl] * s_mul + s_add
                return 0

            lax.fori_loop(0, B // 16, row_body, 0)

        doff = pl.multiple_of(dg * 8, 8)
        pltpu.async_copy(obuf, out_hbm.at[t, pl.ds(doff, 8)], osem)
        pltpu.make_async_copy(obuf, out_hbm.at[t, pl.ds(doff, 8)], osem).wait()
        return 0

    lax.fori_loop(0, ndense, dense_body, 0)

    ncat = jnp.where(wid < 16, 7, 6)

    def cat_body(i, _):
        g = wid + 32 * i
        f = g // 8
        dg = g % 8
        r0 = pl.multiple_of(g * 8, 8)

        koff = pl.multiple_of(f * B, 8)
        pltpu.async_copy(skeys1d_hbm.at[pl.ds(koff, B)], skeys_v, psem)
        boff = pl.multiple_of(f * NBND, 8)
        pltpu.async_copy(bnds1d_hbm.at[pl.ds(boff, NBND)], bnd_v, psem)
        pltpu.make_async_copy(skeys1d_hbm.at[pl.ds(koff, B)], skeys_v,
                              psem).wait()
        pltpu.make_async_copy(bnds1d_hbm.at[pl.ds(boff, NBND)], bnd_v,
                              psem).wait()
        bndlo = bnd_v[pl.ds(0, 16)]
        bndhi = bnd_v[pl.ds(16, 16)]

        def bnd_at(k):
            return bndlo[k] if k < 16 else bndhi[k - 16]

        def win_src(k):
            if k == NWIN - 1:
                return aux_hbm.at[pl.ds(r0, 8)], TAILW
            return (tt_hbm.at[pl.ds(r0, 8), pl.ds(k * W_IDS, W_IDS)],
                    W_IDS)

        src0, w0 = win_src(0)
        pltpu.async_copy(src0, wbuf.at[0, :, pl.ds(0, w0)], ssem)

        for k in range(NWIN):
            src, wk = win_src(k)
            buf = k % 2
            if k + 1 < NWIN:
                src1, wk1 = win_src(k + 1)
                pltpu.async_copy(src1, wbuf.at[1 - buf, :, pl.ds(0, wk1)],
                                 ssem)
            pltpu.make_async_copy(src, wbuf.at[buf, :, pl.ds(0, wk)],
                                  ssem).wait()

            lo = bnd_at(k)
            hi = bnd_at(k + 1)
            lo16 = lax.shift_right_logical(lo, 4)
            nblk = jnp.maximum(
                lax.shift_right_logical(hi + 15, 4) - lo16, 0)
            nblk = jnp.where(hi > lo, nblk, 0)

            def blk_body(jb, _):
                base = (lo16 + jb) * 16
                kv = skeys_v[pl.ds(base, 16)]
                lid = base + lanes
                msk = (lid >= lo) & (lid < hi)
                vv = lax.shift_right_logical(kv, 12) - k * W_IDS
                vv = jnp.clip(vv, 0, jnp.int32(wk - 1))
                pos = lax.bitwise_and(kv, 4095)
                for dd in range(8):
                    dvec = jnp.full((16,), dd, jnp.int32)
                    gval = plsc.load_gather(wbuf.at[buf], [dvec, vv])
                    plsc.store_scatter(obuf, [dvec, pos], gval, mask=msk)
                return 0

            lax.fori_loop(0, nblk, blk_body, 0)

        t = 1 + N_NUM + f
        doff = pl.multiple_of(dg * 8, 8)
        pltpu.async_copy(obuf, out_hbm.at[t, pl.ds(doff, 8)], osem)
        pltpu.make_async_copy(obuf, out_hbm.at[t, pl.ds(doff, 8)], osem).wait()
        return 0

    lax.fori_loop(0, ncat, cat_body, 0)


@jax.jit
def kernel(x_num, x_cat, num_weight, num_bias, cat_tables, cls_token):
    tt = cat_tables.transpose(0, 2, 1).reshape(N_CAT * D, VOCAB)
    aux = jnp.pad(tt[:, TAIL0:], ((0, 0), (0, TAILW - (VOCAB - TAIL0))))
    xcat_t = x_cat.astype(jnp.int32).T
    keys = xcat_t * B + jnp.arange(B, dtype=jnp.int32)[None, :]
    skeys = jnp.sort(keys, axis=1)
    svals = lax.shift_right_logical(skeys, 12)
    wbnd = jnp.arange(NBND - 1, dtype=jnp.int32) * W_IDS
    bnds = jax.vmap(
        lambda row: jnp.searchsorted(row, wbnd, side="left")
    )(svals).astype(jnp.int32)
    bnds = jnp.pad(bnds, ((0, 0), (0, 1)), constant_values=B)
    xn1d = x_num.T.reshape(-1)
    skeys1d = skeys.reshape(-1)
    bnds1d = bnds.reshape(-1)
    w1d = num_weight.reshape(-1)
    b1d = num_bias.reshape(-1)
    cls = cls_token.reshape(D)

    mesh = plsc.VectorSubcoreMesh(core_axis_name="c", subcore_axis_name="s",
                                  num_cores=NUM_CORES,
                                  num_subcores=NUM_SUBCORES)
    run = pl.kernel(
        _tokenizer_kernel,
        out_type=jax.ShapeDtypeStruct((N_TOK, D, B), jnp.float32),
        mesh=mesh,
        scratch_types=[
            pltpu.VMEM((2, 8, W_IDS), jnp.float32),
            pltpu.VMEM((8, B), jnp.float32),
            pltpu.VMEM((B,), jnp.int32),
            pltpu.VMEM((B,), jnp.float32),
            pltpu.VMEM((NBND,), jnp.int32),
            pltpu.VMEM((48,), jnp.float32),
            pltpu.SemaphoreType.DMA,
            pltpu.SemaphoreType.DMA,
            pltpu.SemaphoreType.DMA,
            pltpu.SemaphoreType.DMA,
        ],
        compiler_params=pltpu.CompilerParams(needs_layout_passes=False),
    )
    out_t = run(xn1d, skeys1d, bnds1d, w1d, b1d, cls, tt, aux)
    return out_t.transpose(2, 0, 1)


# speedup vs baseline: 1.5930x; 1.3111x over previous
import jax
import jax.numpy as jnp
from jax import lax
from jax.experimental import pallas as pl
from jax.experimental.pallas import tpu as pltpu
from jax.experimental.pallas import tpu_sc as plsc

N_NUM = 13
N_CAT = 26
VOCAB = 100000
D = 64
B = 4096
N_TOK = 1 + N_NUM + N_CAT

NUM_CORES = 2
NUM_SUBCORES = 16
NW = NUM_CORES * NUM_SUBCORES

W_IDS = 4096
WIN_BASE = [k * W_IDS for k in range(24)] + [24 * W_IDS, 99968]
WIN_W = [W_IDS] * 24 + [1664, 128]
NWIN = len(WIN_BASE)
NBND = 32

N_CGRP = N_CAT * (D // 8)
N_DGRP = (1 + N_NUM) * (D // 8)


def _tokenizer_kernel(xn1d_hbm, skeys1d_hbm, bnds1d_hbm, w1d_hbm, b1d_hbm,
                      cls_hbm, tt_hbm, aux_hbm, out_hbm,
                      wbuf, obuf, skeys_v, xn_v, bnd_v, prm_v,
                      ssem, osem, psem, dsem):
    core = lax.axis_index("c")
    sub = lax.axis_index("s")
    wid = sub * NUM_CORES + core

    lanes = lax.iota(jnp.int32, 16)

    ndense = jnp.where(wid < 16, 3, 4)

    def dense_body(j, _):
        h = (31 - wid) + 32 * j
        t = h // 8
        dg = h % 8
        tm1 = jnp.maximum(t - 1, 0)
        xoff = pl.multiple_of(tm1 * B, 8)
        pltpu.async_copy(xn1d_hbm.at[pl.ds(xoff, B)], xn_v, dsem)
        poff = pl.multiple_of(tm1 * D + dg * 8, 8)
        coff = pl.multiple_of(dg * 8, 8)
        pltpu.async_copy(w1d_hbm.at[pl.ds(poff, 8)], prm_v.at[pl.ds(0, 8)],
                         psem)
        pltpu.async_copy(b1d_hbm.at[pl.ds(poff, 8)], prm_v.at[pl.ds(16, 8)],
                         psem)
        pltpu.async_copy(cls_hbm.at[pl.ds(coff, 8)], prm_v.at[pl.ds(32, 8)],
                         psem)
        pltpu.make_async_copy(xn1d_hbm.at[pl.ds(xoff, B)], xn_v, dsem).wait()
        pltpu.make_async_copy(w1d_hbm.at[pl.ds(poff, 8)],
                              prm_v.at[pl.ds(0, 8)], psem).wait()
        pltpu.make_async_copy(b1d_hbm.at[pl.ds(poff, 8)],
                              prm_v.at[pl.ds(16, 8)], psem).wait()
        pltpu.make_async_copy(cls_hbm.at[pl.ds(coff, 8)],
                              prm_v.at[pl.ds(32, 8)], psem).wait()
        wv8 = prm_v[pl.ds(0, 16)]
        bv8 = prm_v[pl.ds(16, 16)]
        cv8 = prm_v[pl.ds(32, 16)]

        for dd in range(8):
            is_cls = t == 0
            s_mul = jnp.where(is_cls, 0.0, wv8[dd])
            s_add = jnp.where(is_cls, cv8[dd], bv8[dd])

            def row_body(v, _):
                sl = pl.ds(v * 16, 16)
                obuf[dd, sl] = xn_v[sl] * s_mul + s_add
                return 0

            lax.fori_loop(0, B // 16, row_body, 0)

        doff = pl.multiple_of(dg * 8, 8)
        pltpu.async_copy(obuf, out_hbm.at[t, pl.ds(doff, 8)], osem)
        pltpu.make_async_copy(obuf, out_hbm.at[t, pl.ds(doff, 8)], osem).wait()
        return 0

    lax.fori_loop(0, ndense, dense_body, 0)

    ncat = jnp.where(wid < 16, 7, 6)

    def cat_body(i, _):
        g = wid + 32 * i
        f = g // 8
        dg = g % 8
        r0 = pl.multiple_of(g * 8, 8)

        koff = pl.multiple_of(f * B, 8)
        pltpu.async_copy(skeys1d_hbm.at[pl.ds(koff, B)], skeys_v, psem)
        boff = pl.multiple_of(f * NBND, 8)
        pltpu.async_copy(bnds1d_hbm.at[pl.ds(boff, NBND)], bnd_v, psem)
        pltpu.make_async_copy(skeys1d_hbm.at[pl.ds(koff, B)], skeys_v,
                              psem).wait()
        pltpu.make_async_copy(bnds1d_hbm.at[pl.ds(boff, NBND)], bnd_v,
                              psem).wait()
        bndlo = bnd_v[pl.ds(0, 16)]
        bndhi = bnd_v[pl.ds(16, 16)]

        def bnd_at(k):
            return bndlo[k] if k < 16 else bndhi[k - 16]

        def win_src(k):
            if k == NWIN - 1:
                return aux_hbm.at[pl.ds(r0, 8)], WIN_W[k]
            return (tt_hbm.at[pl.ds(r0, 8), pl.ds(WIN_BASE[k], WIN_W[k])],
                    WIN_W[k])

        src0, w0 = win_src(0)
        pltpu.async_copy(src0, wbuf.at[0, :, pl.ds(0, w0)], ssem)

        for k in range(NWIN):
            src, wk = win_src(k)
            buf = k % 2
            if k + 1 < NWIN:
                src1, wk1 = win_src(k + 1)
                pltpu.async_copy(src1, wbuf.at[1 - buf, :, pl.ds(0, wk1)],
                                 ssem)
            pltpu.make_async_copy(src, wbuf.at[buf, :, pl.ds(0, wk)],
                                  ssem).wait()

            lo = bnd_at(k)
            hi = bnd_at(k + 1)
            lo16 = lax.shift_right_logical(lo, 4)
            nblk = jnp.maximum(
                lax.shift_right_logical(hi + 15, 4) - lo16, 0)
            nblk = jnp.where(hi > lo, nblk, 0)

            def blk_body(jb, _):
                base = (lo16 + jb) * 16
                kv = skeys_v[pl.ds(base, 16)]
                lid = base + lanes
                msk = (lid >= lo) & (lid < hi)
                vv = lax.shift_right_logical(kv, 12) - WIN_BASE[k]
                vv = jnp.clip(vv, 0, jnp.int32(wk - 1))
                pos = lax.bitwise_and(kv, 4095)
                for dd in range(8):
                    dvec = jnp.full((16,), dd, jnp.int32)
                    gval = plsc.load_gather(wbuf.at[buf], [dvec, vv])
                    plsc.store_scatter(obuf, [dvec, pos], gval, mask=msk)
                return 0

            lax.fori_loop(0, nblk, blk_body, 0)

        t = 1 + N_NUM + f
        doff = pl.multiple_of(dg * 8, 8)
        pltpu.async_copy(obuf, out_hbm.at[t, pl.ds(doff, 8)], osem)
        pltpu.make_async_copy(obuf, out_hbm.at[t, pl.ds(doff, 8)], osem).wait()
        return 0

    lax.fori_loop(0, ncat, cat_body, 0)


@jax.jit
def kernel(x_num, x_cat, num_weight, num_bias, cat_tables, cls_token):
    tt = cat_tables.transpose(0, 2, 1).reshape(N_CAT * D, VOCAB)
    aux = jnp.pad(tt[:, WIN_BASE[-1]:], ((0, 0), (0, 96)))
    xcat_t = x_cat.astype(jnp.int32).T
    keys = xcat_t * B + jnp.arange(B, dtype=jnp.int32)[None, :]
    skeys = jnp.sort(keys, axis=1)
    bvals = jnp.array(WIN_BASE + [VOCAB], dtype=jnp.int32)
    bnds = jnp.sum(xcat_t[:, :, None] < bvals[None, None, :],
                   axis=1, dtype=jnp.int32)
    bnds = jnp.pad(bnds, ((0, 0), (0, NBND - bnds.shape[1])))
    xn1d = x_num.T.reshape(-1)
    skeys1d = skeys.reshape(-1)
    bnds1d = bnds.reshape(-1)
    w1d = num_weight.reshape(-1)
    b1d = num_bias.reshape(-1)
    cls = cls_token.reshape(D)

    mesh = plsc.VectorSubcoreMesh(core_axis_name="c", subcore_axis_name="s",
                                  num_cores=NUM_CORES,
                                  num_subcores=NUM_SUBCORES)
    run = pl.kernel(
        _tokenizer_kernel,
        out_type=jax.ShapeDtypeStruct((N_TOK, D, B), jnp.float32),
        mesh=mesh,
        scratch_types=[
            pltpu.VMEM((2, 8, W_IDS), jnp.float32),
            pltpu.VMEM((8, B), jnp.float32),
            pltpu.VMEM((B,), jnp.int32),
            pltpu.VMEM((B,), jnp.float32),
            pltpu.VMEM((NBND,), jnp.int32),
            pltpu.VMEM((48,), jnp.float32),
            pltpu.SemaphoreType.DMA,
            pltpu.SemaphoreType.DMA,
            pltpu.SemaphoreType.DMA,
            pltpu.SemaphoreType.DMA,
        ],
        compiler_params=pltpu.CompilerParams(needs_layout_passes=False),
    )
    out_t = run(xn1d, skeys1d, bnds1d, w1d, b1d, cls, tt, aux)
    return out_t.transpose(2, 0, 1)
